# baseline (speedup 1.0000x reference)
import jax
import jax.numpy as jnp
from jax.experimental import pallas as pl
from jax.experimental.pallas import tpu as pltpu

_N, _C = 32768, 100
_R = 2048


def _tc_body(x_ref, w_ref, b_ref, o_ref):
    x = x_ref[...]
    w = w_ref[...]
    lane = jax.lax.broadcasted_iota(jnp.int32, x.shape, 1)
    ok = jnp.logical_and(lane < _C, jnp.logical_not(jnp.isnan(x)))
    contrib = jnp.where(ok, x * w, jnp.float32(0.0))
    o_ref[...] = jnp.sum(contrib, axis=1, keepdims=True) + b_ref[0, 0]


def kernel(local_map_predictions, weights_pool, bias):
    x = local_map_predictions
    w2 = weights_pool[None, :]
    b2 = bias[None, :]
    out = pl.pallas_call(
        _tc_body,
        grid=(_N // _R,),
        in_specs=[
            pl.BlockSpec((_R, _C), lambda i: (i, 0)),
            pl.BlockSpec((1, _C), lambda i: (0, 0)),
            pl.BlockSpec(memory_space=pltpu.SMEM),
        ],
        out_specs=pl.BlockSpec((_R, 1), lambda i: (i, 0)),
        out_shape=jax.ShapeDtypeStruct((_N, 1), jnp.float32),
    )(x, w2, b2)
    return out


# speedup vs baseline: 1.3070x; 1.3070x over previous
import jax
import jax.numpy as jnp
from jax.experimental import pallas as pl
from jax.experimental.pallas import tpu as pltpu

_N, _C = 32768, 100
_R = 4096


def _tc_body(x_ref, w_ref, b_ref, o_ref):
    t = x_ref[...] * w_ref[...]
    contrib = jnp.where(t != t, jnp.float32(0.0), t)
    o_ref[...] = jnp.sum(contrib, axis=1) + b_ref[0]


def kernel(local_map_predictions, weights_pool, bias):
    x = local_map_predictions
    w2 = weights_pool[None, :]
    out = pl.pallas_call(
        _tc_body,
        grid=(_N // _R,),
        in_specs=[
            pl.BlockSpec((_R, _C), lambda i: (i, 0)),
            pl.BlockSpec((1, _C), lambda i: (0, 0)),
            pl.BlockSpec(memory_space=pltpu.SMEM),
        ],
        out_specs=pl.BlockSpec((_R,), lambda i: (i,)),
        out_shape=jax.ShapeDtypeStruct((_N,), jnp.float32),
    )(x, w2, bias)
    return out[:, None]
